# baseline (speedup 1.0000x reference)
import jax
import jax.numpy as jnp
from jax.experimental import pallas as pl

K = 20


def _knn(x, k):
    xt = jnp.swapaxes(x, 2, 1)
    inner = -2.0 * jnp.matmul(xt, x)
    xx = jnp.sum(x * x, axis=1, keepdims=True)
    pairwise = -xx - inner - jnp.swapaxes(xx, 2, 1)
    _, idx = jax.lax.top_k(pairwise, k)
    return idx


def _bn_relu_body(y_ref, s_ref, t_ref, o_ref):
    o_ref[...] = jnp.maximum(y_ref[...] * s_ref[...] + t_ref[...], 0.0)


def kernel(x, W0, b0, gamma0, beta0):
    B, C, N = x.shape
    k = K
    idx = _knn(x, k)
    idx_base = jnp.arange(B, dtype=idx.dtype).reshape(-1, 1, 1) * N
    idx_flat = (idx + idx_base).reshape(-1)
    xt = jnp.swapaxes(x, 2, 1)
    flat = xt.reshape(B * N, C)
    feature = jnp.take(flat, idx_flat, axis=0).reshape(B, N, k, C)
    center = jnp.broadcast_to(xt[:, :, None, :], (B, N, k, C))
    feat = jnp.concatenate([feature - center, center], axis=3)
    feat = jnp.transpose(feat, (0, 3, 1, 2))
    y = jnp.einsum('oc,bcnk->bonk', W0, feat) + b0[None, :, None, None]
    mean = jnp.mean(y, axis=(0, 2, 3), keepdims=True)
    var = jnp.var(y, axis=(0, 2, 3), keepdims=True)
    inv = 1.0 / jnp.sqrt(var + 1e-5)
    scale = (gamma0[None, :, None, None] * inv)
    shift = (beta0[None, :, None, None] - mean * scale)
    O = 64
    yf = y.reshape(B, O, N * k)
    sf = scale.reshape(1, O, 1)
    tf = shift.reshape(1, O, 1)
    out = pl.pallas_call(
        _bn_relu_body,
        grid=(B, 4),
        in_specs=[
            pl.BlockSpec((1, O // 4, N * k), lambda b, o: (b, o, 0)),
            pl.BlockSpec((1, O // 4, 1), lambda b, o: (0, o, 0)),
            pl.BlockSpec((1, O // 4, 1), lambda b, o: (0, o, 0)),
        ],
        out_specs=pl.BlockSpec((1, O // 4, N * k), lambda b, o: (b, o, 0)),
        out_shape=jax.ShapeDtypeStruct((B, O, N * k), jnp.float32),
    )(yf, sf, tf)
    return out.reshape(B, O, N, k)


# speedup vs baseline: 5.1586x; 5.1586x over previous
import functools

import jax
import jax.numpy as jnp
from jax import lax
from jax.experimental import pallas as pl
from jax.experimental.pallas import tpu as pltpu
from jax.experimental.pallas import tpu_sc as plsc

KK = 20
TN = 256

NC = 2
NS = 16
NW = NC * NS


def _tc_body(x_ref, wa_ref, wd_ref, b0_ref, z_ref, c_ref, idx_ref):
    nt = pl.program_id(1)
    xb = x_ref[0]
    C, N = xb.shape

    @pl.when(nt == 0)
    def _():
        z_ref[0] = jnp.dot(wa_ref[...], xb, preferred_element_type=jnp.float32)
        c_ref[0] = (jnp.dot(wd_ref[...], xb, preferred_element_type=jnp.float32)
                    + b0_ref[...].reshape(C, 1))

    xt = x_ref[0, :, pl.ds(nt * TN, TN)]
    inner = lax.dot_general(xt, xb, (((0,), (0,)), ((), ())),
                            preferred_element_type=jnp.float32)
    xx = jnp.sum(xb * xb, axis=0)
    xx_rows = jnp.sum(xt * xt, axis=0)
    d = 2.0 * inner - xx[None, :] - xx_rows[:, None]

    iota = lax.broadcasted_iota(jnp.int32, (TN, N), 1)
    neg = jnp.float32(-jnp.inf)
    for j in range(KK):
        m = jnp.max(d, axis=1, keepdims=True)
        eq = d == m
        idxj = jnp.min(jnp.where(eq, iota, N), axis=1)
        idx_ref[0, j, :] = idxj
        if j + 1 < KK:
            d = jnp.where(eq, neg, d)


def _tc_stage(x, W0, b0):
    B, C, N = x.shape
    Wa = W0[:, :C]
    Wd = W0[:, C:] - W0[:, :C]
    grid = (B, N // TN)
    z, c, idxT = pl.pallas_call(
        _tc_body,
        grid=grid,
        in_specs=[
            pl.BlockSpec((1, C, N), lambda b, t: (b, 0, 0)),
            pl.BlockSpec((C, 2 * C // 2), lambda b, t: (0, 0)),
            pl.BlockSpec((C, 2 * C // 2), lambda b, t: (0, 0)),
            pl.BlockSpec((1, C), lambda b, t: (0, 0)),
        ],
        out_specs=[
            pl.BlockSpec((1, C, N), lambda b, t: (b, 0, 0)),
            pl.BlockSpec((1, C, N), lambda b, t: (b, 0, 0)),
            pl.BlockSpec((1, KK, TN), lambda b, t: (b, 0, t)),
        ],
        out_shape=[
            jax.ShapeDtypeStruct((B, C, N), jnp.float32),
            jax.ShapeDtypeStruct((B, C, N), jnp.float32),
            jax.ShapeDtypeStruct((B, KK, N), jnp.int32),
        ],
    )(x, Wa, Wd, b0.reshape(1, C))
    return z, c, idxT


def _make_sc_stats(B, C, N):
    mesh = plsc.VectorSubcoreMesh(core_axis_name="c", subcore_axis_name="s")
    o_per_w = B * C // NW

    @functools.partial(
        pl.kernel,
        out_type=jax.ShapeDtypeStruct((B, C, 2, 16), jnp.float32),
        mesh=mesh,
        compiler_params=pltpu.CompilerParams(needs_layout_passes=False),
        scratch_types=[
            pltpu.VMEM((KK, N), jnp.int32),
            pltpu.VMEM((N,), jnp.float32),
            pltpu.VMEM((N,), jnp.float32),
            pltpu.VMEM((2, 16), jnp.float32),
        ],
    )
    def stats_k(z_hbm, c_hbm, idx_hbm, out_hbm, idx_v, z_v, c_v, st_v):
        wid = lax.axis_index("s") * NC + lax.axis_index("c")
        b = wid // (C // o_per_w)
        o0 = (wid % (C // o_per_w)) * o_per_w
        pltpu.sync_copy(idx_hbm.at[b], idx_v)

        def otask(i, carry):
            o = o0 + i
            pltpu.sync_copy(z_hbm.at[b, o], z_v)
            pltpu.sync_copy(c_hbm.at[b, o], c_v)

            def chunk(nc, accs):
                a1, a2, ax, ac, acs = accs
                s = pl.ds(nc * 16, 16)
                cvec = c_v[s]
                vs = jnp.zeros((16,), jnp.float32)
                for j in range(KK):
                    iv = idx_v[j, s]
                    g = plsc.load_gather(z_v, [iv])
                    vs = vs + g
                    a2 = a2 + g * g
                a1 = a1 + vs
                ax = ax + cvec * vs
                ac = ac + cvec
                acs = acs + cvec * cvec
                return a1, a2, ax, ac, acs

            zv16 = jnp.zeros((16,), jnp.float32)
            a1, a2, ax, ac, acs = lax.fori_loop(
                0, N // 16, chunk, (zv16, zv16, zv16, zv16, zv16))
            st_v[0, :] = a1 + jnp.float32(KK) * ac
            st_v[1, :] = a2 + 2.0 * ax + jnp.float32(KK) * acs
            pltpu.sync_copy(st_v, out_hbm.at[b, o])
            return carry

        lax.fori_loop(0, o_per_w, otask, 0)

    return stats_k


def _make_sc_out(B, C, N):
    mesh = plsc.VectorSubcoreMesh(core_axis_name="c", subcore_axis_name="s")
    o_per_w = B * C // NW
    NKH = N * KK // 2

    @functools.partial(
        pl.kernel,
        out_type=jax.ShapeDtypeStruct((B, C, N * KK), jnp.float32),
        mesh=mesh,
        compiler_params=pltpu.CompilerParams(needs_layout_passes=False),
        scratch_types=[
            pltpu.VMEM((KK, N), jnp.int32),
            pltpu.VMEM((N,), jnp.float32),
            pltpu.VMEM((N,), jnp.float32),
            pltpu.VMEM((NKH,), jnp.float32),
            pltpu.VMEM((NKH,), jnp.float32),
            pltpu.VMEM((2 * C,), jnp.float32),
            pltpu.SemaphoreType.DMA,
            pltpu.SemaphoreType.DMA,
        ],
    )
    def out_k(z_hbm, c_hbm, idx_hbm, ad_hbm, out_hbm,
              idx_v, zs_v, u_v, ob0_v, ob1_v, ad_v, sem0, sem1):
        wid = lax.axis_index("s") * NC + lax.axis_index("c")
        b = wid // (C // o_per_w)
        o0 = (wid % (C // o_per_w)) * o_per_w
        pltpu.sync_copy(idx_hbm.at[b], idx_v)
        pltpu.sync_copy(ad_hbm, ad_v)
        lane20 = lax.iota(jnp.int32, 16) * KK

        def otask(i, carry):
            o = o0 + i
            pltpu.sync_copy(z_hbm.at[b, o], zs_v)
            pltpu.sync_copy(c_hbm.at[b, o], u_v)
            a = plsc.load_gather(ad_v, [jnp.full((16,), o, jnp.int32)])
            dd = plsc.load_gather(ad_v, [jnp.full((16,), C + o, jnp.int32)])

            def scale_chunk(nc, carry2):
                s = pl.ds(nc * 16, 16)
                zs_v[s] = zs_v[s] * a
                u_v[s] = u_v[s] * a + dd
                return carry2

            lax.fori_loop(0, N // 16, scale_chunk, 0)

            for half, (ob, sem) in enumerate(((ob0_v, sem0), (ob1_v, sem1))):
                def chunk(nc, carry2, half=half, ob=ob):
                    n0 = half * (N // 2) + nc * 16
                    uvec = u_v[pl.ds(n0, 16)]
                    base = nc * (16 * KK)
                    for j in range(KK):
                        iv = idx_v[j, pl.ds(n0, 16)]
                        g = plsc.load_gather(zs_v, [iv])
                        y = jnp.maximum(g + uvec, 0.0)
                        pos = lane20 + (base + j)
                        plsc.store_scatter(ob, [pos], y)
                    return carry2

                lax.fori_loop(0, N // 32, chunk, 0)
                pltpu.async_copy(
                    ob, out_hbm.at[b, o, pl.ds(half * NKH, NKH)], sem)
            pltpu.make_async_copy(
                ob0_v, out_hbm.at[b, o, pl.ds(0, NKH)], sem0).wait()
            pltpu.make_async_copy(
                ob1_v, out_hbm.at[b, o, pl.ds(NKH, NKH)], sem1).wait()
            return carry

        lax.fori_loop(0, o_per_w, otask, 0)

    return out_k


def kernel(x, W0, b0, gamma0, beta0):
    B, C, N = x.shape
    z, c, idxT = _tc_stage(x, W0, b0)

    stats = _make_sc_stats(B, C, N)(z, c, idxT)
    cnt = jnp.float32(B * N * KK)
    sum_y = jnp.sum(stats[:, :, 0, :], axis=(0, 2))
    sum_y2 = jnp.sum(stats[:, :, 1, :], axis=(0, 2))
    mean = sum_y / cnt
    var = sum_y2 / cnt - mean * mean
    a = gamma0 / jnp.sqrt(var + 1e-5)
    d = beta0 - mean * a
    ad = jnp.concatenate([a, d])

    out = _make_sc_out(B, C, N)(z, c, idxT, ad)
    return out.reshape(B, C, N, KK)


# speedup vs baseline: 6.8354x; 1.3251x over previous
import functools

import jax
import jax.numpy as jnp
from jax import lax
from jax.experimental import pallas as pl
from jax.experimental.pallas import tpu as pltpu
from jax.experimental.pallas import tpu_sc as plsc

KK = 20
TN = 256

NC = 2
NS = 16
NW = NC * NS


def _tc_body(x_ref, wa_ref, wd_ref, b0_ref, z_ref, c_ref, idx_ref):
    nt = pl.program_id(1)
    xb = x_ref[0]
    C, N = xb.shape

    @pl.when(nt == 0)
    def _():
        z_ref[0] = jnp.dot(wa_ref[...], xb, preferred_element_type=jnp.float32)
        c_ref[0] = (jnp.dot(wd_ref[...], xb, preferred_element_type=jnp.float32)
                    + b0_ref[...].reshape(C, 1))

    xt = x_ref[0, :, pl.ds(nt * TN, TN)]
    inner = lax.dot_general(xt, xb, (((0,), (0,)), ((), ())),
                            preferred_element_type=jnp.float32)
    xx = jnp.sum(xb * xb, axis=0)
    xx_rows = jnp.sum(xt * xt, axis=0)
    d = ((-xx_rows[:, None]) - (-2.0 * inner)) - xx[None, :]

    iota = lax.broadcasted_iota(jnp.int32, (TN, N), 1)
    neg = jnp.float32(-jnp.inf)
    for j in range(KK):
        m = jnp.max(d, axis=1, keepdims=True)
        eq = d == m
        idxj = jnp.min(jnp.where(eq, iota, N), axis=1)
        idx_ref[0, j, :] = idxj
        if j + 1 < KK:
            d = jnp.where(eq, neg, d)


def _tc_stage(x, W0, b0):
    B, C, N = x.shape
    Wa = W0[:, :C]
    Wd = W0[:, C:] - W0[:, :C]
    grid = (B, N // TN)
    z, c, idxT = pl.pallas_call(
        _tc_body,
        grid=grid,
        in_specs=[
            pl.BlockSpec((1, C, N), lambda b, t: (b, 0, 0)),
            pl.BlockSpec((C, 2 * C // 2), lambda b, t: (0, 0)),
            pl.BlockSpec((C, 2 * C // 2), lambda b, t: (0, 0)),
            pl.BlockSpec((1, C), lambda b, t: (0, 0)),
        ],
        out_specs=[
            pl.BlockSpec((1, C, N), lambda b, t: (b, 0, 0)),
            pl.BlockSpec((1, C, N), lambda b, t: (b, 0, 0)),
            pl.BlockSpec((1, KK, TN), lambda b, t: (b, 0, t)),
        ],
        out_shape=[
            jax.ShapeDtypeStruct((B, C, N), jnp.float32),
            jax.ShapeDtypeStruct((B, C, N), jnp.float32),
            jax.ShapeDtypeStruct((B, KK, N), jnp.int32),
        ],
    )(x, Wa, Wd, b0.reshape(1, C))
    return z, c, idxT


def _make_sc_stats(B, C, N):
    mesh = plsc.VectorSubcoreMesh(core_axis_name="c", subcore_axis_name="s")
    o_per_w = B * C // NW

    @functools.partial(
        pl.kernel,
        out_type=jax.ShapeDtypeStruct((B, C, 2, 16), jnp.float32),
        mesh=mesh,
        compiler_params=pltpu.CompilerParams(needs_layout_passes=False),
        scratch_types=[
            pltpu.VMEM((KK, N), jnp.int32),
            pltpu.VMEM((N,), jnp.float32),
            pltpu.VMEM((N,), jnp.float32),
            pltpu.VMEM((2, 16), jnp.float32),
        ],
    )
    def stats_k(z_hbm, c_hbm, idx_hbm, out_hbm, idx_v, z_v, c_v, st_v):
        wid = lax.axis_index("s") * NC + lax.axis_index("c")
        b = wid // (C // o_per_w)
        o0 = (wid % (C // o_per_w)) * o_per_w
        pltpu.sync_copy(idx_hbm.at[b], idx_v)

        def otask(i, carry):
            o = o0 + i
            pltpu.sync_copy(z_hbm.at[b, o], z_v)
            pltpu.sync_copy(c_hbm.at[b, o], c_v)

            def chunk(nc, accs):
                a1, a2, ax, ac, acs = accs
                s = pl.ds(nc * 16, 16)
                cvec = c_v[s]
                vs = jnp.zeros((16,), jnp.float32)
                for j in range(KK):
                    iv = idx_v[j, s]
                    g = plsc.load_gather(z_v, [iv])
                    vs = vs + g
                    a2 = a2 + g * g
                a1 = a1 + vs
                ax = ax + cvec * vs
                ac = ac + cvec
                acs = acs + cvec * cvec
                return a1, a2, ax, ac, acs

            zv16 = jnp.zeros((16,), jnp.float32)
            a1, a2, ax, ac, acs = lax.fori_loop(
                0, N // 16, chunk, (zv16, zv16, zv16, zv16, zv16))
            st_v[0, :] = a1 + jnp.float32(KK) * ac
            st_v[1, :] = a2 + 2.0 * ax + jnp.float32(KK) * acs
            pltpu.sync_copy(st_v, out_hbm.at[b, o])
            return carry

        lax.fori_loop(0, o_per_w, otask, 0)

    return stats_k


def _make_sc_out(B, C, N):
    mesh = plsc.VectorSubcoreMesh(core_axis_name="c", subcore_axis_name="s")
    o_per_w = B * C // NW
    NKH = N * KK // 2

    @functools.partial(
        pl.kernel,
        out_type=jax.ShapeDtypeStruct((B, C, N * KK), jnp.float32),
        mesh=mesh,
        compiler_params=pltpu.CompilerParams(needs_layout_passes=False),
        scratch_types=[
            pltpu.VMEM((KK, N), jnp.int32),
            pltpu.VMEM((N,), jnp.float32),
            pltpu.VMEM((N,), jnp.float32),
            pltpu.VMEM((NKH,), jnp.float32),
            pltpu.VMEM((NKH,), jnp.float32),
            pltpu.VMEM((2 * C,), jnp.float32),
            pltpu.SemaphoreType.DMA,
            pltpu.SemaphoreType.DMA,
        ],
    )
    def out_k(z_hbm, c_hbm, idx_hbm, ad_hbm, out_hbm,
              idx_v, zs_v, u_v, ob0_v, ob1_v, ad_v, sem0, sem1):
        wid = lax.axis_index("s") * NC + lax.axis_index("c")
        b = wid // (C // o_per_w)
        o0 = (wid % (C // o_per_w)) * o_per_w
        pltpu.sync_copy(idx_hbm.at[b], idx_v)
        pltpu.sync_copy(ad_hbm, ad_v)
        lane20 = lax.iota(jnp.int32, 16) * KK

        def otask(i, carry):
            o = o0 + i
            pltpu.sync_copy(z_hbm.at[b, o], zs_v)
            pltpu.sync_copy(c_hbm.at[b, o], u_v)
            a = plsc.load_gather(ad_v, [jnp.full((16,), o, jnp.int32)])
            dd = plsc.load_gather(ad_v, [jnp.full((16,), C + o, jnp.int32)])

            @plsc.parallel_loop(0, N // 16)
            def _(nc):
                s = pl.ds(nc * 16, 16)
                zs_v[s] = zs_v[s] * a
                u_v[s] = u_v[s] * a + dd

            for half, (ob, sem) in enumerate(((ob0_v, sem0), (ob1_v, sem1))):
                @pl.when(i > 0)
                def _(ob=ob, sem=sem):
                    pltpu.make_async_copy(
                        ob, out_hbm.at[b, o, pl.ds(half * NKH, NKH)],
                        sem).wait()

                @plsc.parallel_loop(0, N // 32)
                def _(nc, half=half, ob=ob):
                    n0 = half * (N // 2) + nc * 16
                    uvec = u_v[pl.ds(n0, 16)]
                    base = nc * (16 * KK)
                    for j in range(KK):
                        iv = idx_v[j, pl.ds(n0, 16)]
                        g = plsc.load_gather(zs_v, [iv])
                        y = jnp.maximum(g + uvec, 0.0)
                        pos = lane20 + (base + j)
                        plsc.store_scatter(ob, [pos], y)

                pltpu.async_copy(
                    ob, out_hbm.at[b, o, pl.ds(half * NKH, NKH)], sem)
            return carry

        lax.fori_loop(0, o_per_w, otask, 0)
        o_last = o0 + o_per_w - 1
        pltpu.make_async_copy(
            ob0_v, out_hbm.at[b, o_last, pl.ds(0, NKH)], sem0).wait()
        pltpu.make_async_copy(
            ob1_v, out_hbm.at[b, o_last, pl.ds(NKH, NKH)], sem1).wait()

    return out_k


def kernel(x, W0, b0, gamma0, beta0):
    B, C, N = x.shape
    z, c, idxT = _tc_stage(x, W0, b0)

    stats = _make_sc_stats(B, C, N)(z, c, idxT)
    cnt = jnp.float32(B * N * KK)
    sum_y = jnp.sum(stats[:, :, 0, :], axis=(0, 2))
    sum_y2 = jnp.sum(stats[:, :, 1, :], axis=(0, 2))
    mean = sum_y / cnt
    var = sum_y2 / cnt - mean * mean
    a = gamma0 / jnp.sqrt(var + 1e-5)
    d = beta0 - mean * a
    ad = jnp.concatenate([a, d])

    out = _make_sc_out(B, C, N)(z, c, idxT, ad)
    return out.reshape(B, C, N, KK)


# speedup vs baseline: 7.2461x; 1.0601x over previous
import functools

import jax
import jax.numpy as jnp
from jax import lax
from jax.experimental import pallas as pl
from jax.experimental.pallas import tpu as pltpu
from jax.experimental.pallas import tpu_sc as plsc

KK = 20
TN = 256

NC = 2
NS = 16
NW = NC * NS


def _tc_body(x_ref, wa_ref, wd_ref, b0_ref, z_ref, c_ref, idx_ref):
    nt = pl.program_id(1)
    xb = x_ref[0]
    C, N = xb.shape

    @pl.when(nt == 0)
    def _():
        z_ref[0] = jnp.dot(wa_ref[...], xb, preferred_element_type=jnp.float32)
        c_ref[0] = (jnp.dot(wd_ref[...], xb, preferred_element_type=jnp.float32)
                    + b0_ref[...].reshape(C, 1))

    xt = x_ref[0, :, pl.ds(nt * TN, TN)]
    inner = lax.dot_general(xt, xb, (((0,), (0,)), ((), ())),
                            preferred_element_type=jnp.float32)
    xx = jnp.sum(xb * xb, axis=0)
    xx_rows = jnp.sum(xt * xt, axis=0)
    d = ((-xx_rows[:, None]) - (-2.0 * inner)) - xx[None, :]

    iota = lax.broadcasted_iota(jnp.int32, (TN, N), 1)
    neg = jnp.float32(-jnp.inf)
    for j in range(KK):
        idxj = jnp.argmax(d, axis=1).astype(jnp.int32)
        idx_ref[0, j, :] = idxj
        if j + 1 < KK:
            d = jnp.where(iota == idxj[:, None], neg, d)


def _tc_stage(x, W0, b0):
    B, C, N = x.shape
    Wa = W0[:, :C]
    Wd = W0[:, C:] - W0[:, :C]
    grid = (B, N // TN)
    z, c, idxT = pl.pallas_call(
        _tc_body,
        grid=grid,
        in_specs=[
            pl.BlockSpec((1, C, N), lambda b, t: (b, 0, 0)),
            pl.BlockSpec((C, 2 * C // 2), lambda b, t: (0, 0)),
            pl.BlockSpec((C, 2 * C // 2), lambda b, t: (0, 0)),
            pl.BlockSpec((1, C), lambda b, t: (0, 0)),
        ],
        out_specs=[
            pl.BlockSpec((1, C, N), lambda b, t: (b, 0, 0)),
            pl.BlockSpec((1, C, N), lambda b, t: (b, 0, 0)),
            pl.BlockSpec((1, KK, TN), lambda b, t: (b, 0, t)),
        ],
        out_shape=[
            jax.ShapeDtypeStruct((B, C, N), jnp.float32),
            jax.ShapeDtypeStruct((B, C, N), jnp.float32),
            jax.ShapeDtypeStruct((B, KK, N), jnp.int32),
        ],
    )(x, Wa, Wd, b0.reshape(1, C))
    return z, c, idxT


def _make_sc_stats(B, C, N):
    mesh = plsc.VectorSubcoreMesh(core_axis_name="c", subcore_axis_name="s")
    o_per_w = B * C // NW

    @functools.partial(
        pl.kernel,
        out_type=jax.ShapeDtypeStruct((B, C, 2, 16), jnp.float32),
        mesh=mesh,
        compiler_params=pltpu.CompilerParams(needs_layout_passes=False),
        scratch_types=[
            pltpu.VMEM((KK, N), jnp.int32),
            pltpu.VMEM((N,), jnp.float32),
            pltpu.VMEM((N,), jnp.float32),
            pltpu.VMEM((2, 16), jnp.float32),
        ],
    )
    def stats_k(z_hbm, c_hbm, idx_hbm, out_hbm, idx_v, z_v, c_v, st_v):
        wid = lax.axis_index("s") * NC + lax.axis_index("c")
        b = wid // (C // o_per_w)
        o0 = (wid % (C // o_per_w)) * o_per_w
        pltpu.sync_copy(idx_hbm.at[b], idx_v)

        def otask(i, carry):
            o = o0 + i
            pltpu.sync_copy(z_hbm.at[b, o], z_v)
            pltpu.sync_copy(c_hbm.at[b, o], c_v)

            def chunk(nc, accs):
                a1, a2, ax, ac, acs = accs
                s = pl.ds(nc * 16, 16)
                cvec = c_v[s]
                vs = jnp.zeros((16,), jnp.float32)
                for j in range(KK):
                    iv = idx_v[j, s]
                    g = plsc.load_gather(z_v, [iv])
                    vs = vs + g
                    a2 = a2 + g * g
                a1 = a1 + vs
                ax = ax + cvec * vs
                ac = ac + cvec
                acs = acs + cvec * cvec
                return a1, a2, ax, ac, acs

            zv16 = jnp.zeros((16,), jnp.float32)
            a1, a2, ax, ac, acs = lax.fori_loop(
                0, N // 16, chunk, (zv16, zv16, zv16, zv16, zv16))
            st_v[0, :] = a1 + jnp.float32(KK) * ac
            st_v[1, :] = a2 + 2.0 * ax + jnp.float32(KK) * acs
            pltpu.sync_copy(st_v, out_hbm.at[b, o])
            return carry

        lax.fori_loop(0, o_per_w, otask, 0)

    return stats_k


def _make_sc_out(B, C, N):
    mesh = plsc.VectorSubcoreMesh(core_axis_name="c", subcore_axis_name="s")
    o_per_w = B * C // NW
    NKH = N * KK // 2

    @functools.partial(
        pl.kernel,
        out_type=jax.ShapeDtypeStruct((B, C, N * KK), jnp.float32),
        mesh=mesh,
        compiler_params=pltpu.CompilerParams(needs_layout_passes=False),
        scratch_types=[
            pltpu.VMEM((KK, N), jnp.int32),
            pltpu.VMEM((N,), jnp.float32),
            pltpu.VMEM((N,), jnp.float32),
            pltpu.VMEM((NKH,), jnp.float32),
            pltpu.VMEM((NKH,), jnp.float32),
            pltpu.VMEM((2 * C,), jnp.float32),
            pltpu.SemaphoreType.DMA,
            pltpu.SemaphoreType.DMA,
        ],
    )
    def out_k(z_hbm, c_hbm, idx_hbm, ad_hbm, out_hbm,
              idx_v, zs_v, u_v, ob0_v, ob1_v, ad_v, sem0, sem1):
        wid = lax.axis_index("s") * NC + lax.axis_index("c")
        b = wid // (C // o_per_w)
        o0 = (wid % (C // o_per_w)) * o_per_w
        pltpu.sync_copy(idx_hbm.at[b], idx_v)
        pltpu.sync_copy(ad_hbm, ad_v)
        lane20 = lax.iota(jnp.int32, 16) * KK

        def otask(i, carry):
            o = o0 + i
            pltpu.sync_copy(z_hbm.at[b, o], zs_v)
            pltpu.sync_copy(c_hbm.at[b, o], u_v)
            a = plsc.load_gather(ad_v, [jnp.full((16,), o, jnp.int32)])
            dd = plsc.load_gather(ad_v, [jnp.full((16,), C + o, jnp.int32)])

            @plsc.parallel_loop(0, N // 16)
            def _(nc):
                s = pl.ds(nc * 16, 16)
                zs_v[s] = zs_v[s] * a
                u_v[s] = u_v[s] * a + dd

            for half, (ob, sem) in enumerate(((ob0_v, sem0), (ob1_v, sem1))):
                @pl.when(i > 0)
                def _(ob=ob, sem=sem):
                    pltpu.make_async_copy(
                        ob, out_hbm.at[b, o, pl.ds(half * NKH, NKH)],
                        sem).wait()

                @plsc.parallel_loop(0, N // 32)
                def _(nc, half=half, ob=ob):
                    n0 = half * (N // 2) + nc * 16
                    uvec = u_v[pl.ds(n0, 16)]
                    base = nc * (16 * KK)
                    for j in range(KK):
                        iv = idx_v[j, pl.ds(n0, 16)]
                        g = plsc.load_gather(zs_v, [iv])
                        y = jnp.maximum(g + uvec, 0.0)
                        pos = lane20 + (base + j)
                        plsc.store_scatter(ob, [pos], y)

                pltpu.async_copy(
                    ob, out_hbm.at[b, o, pl.ds(half * NKH, NKH)], sem)
            return carry

        lax.fori_loop(0, o_per_w, otask, 0)
        o_last = o0 + o_per_w - 1
        pltpu.make_async_copy(
            ob0_v, out_hbm.at[b, o_last, pl.ds(0, NKH)], sem0).wait()
        pltpu.make_async_copy(
            ob1_v, out_hbm.at[b, o_last, pl.ds(NKH, NKH)], sem1).wait()

    return out_k


def kernel(x, W0, b0, gamma0, beta0):
    B, C, N = x.shape
    z, c, idxT = _tc_stage(x, W0, b0)

    stats = _make_sc_stats(B, C, N)(z, c, idxT)
    cnt = jnp.float32(B * N * KK)
    sum_y = jnp.sum(stats[:, :, 0, :], axis=(0, 2))
    sum_y2 = jnp.sum(stats[:, :, 1, :], axis=(0, 2))
    mean = sum_y / cnt
    var = sum_y2 / cnt - mean * mean
    a = gamma0 / jnp.sqrt(var + 1e-5)
    d = beta0 - mean * a
    ad = jnp.concatenate([a, d])

    out = _make_sc_out(B, C, N)(z, c, idxT, ad)
    return out.reshape(B, C, N, KK)
